# initial kernel scaffold (unmeasured)
import jax
import jax.numpy as jnp
from jax import lax
from jax.experimental import pallas as pl
from jax.experimental.pallas import tpu as pltpu

N_DEV = 4


def kernel(A, B):
    m, k = A.shape
    _, n = B.shape

    def body(a_ref, b_ref, out_ref, comm_ref, send_sems, recv_sems):
        my = lax.axis_index("i")
        left = lax.rem(my + N_DEV - 1, N_DEV)
        right = lax.rem(my + 1, N_DEV)

        barrier_sem = pltpu.get_barrier_semaphore()
        for nbr in (left, right):
            pl.semaphore_signal(
                barrier_sem, inc=1,
                device_id=(nbr,), device_id_type=pl.DeviceIdType.MESH,
            )
        pl.semaphore_wait(barrier_sem, 2)

        p = jnp.dot(
            a_ref[...].astype(jnp.bfloat16),
            b_ref[...].astype(jnp.bfloat16),
            preferred_element_type=jnp.float32,
        )
        out_ref[...] = p
        comm_ref[0, :, :] = p.astype(jnp.bfloat16)

        for h in range(N_DEV - 1):
            s, r = h % 2, (h + 1) % 2
            rdma = pltpu.make_async_remote_copy(
                src_ref=comm_ref.at[s],
                dst_ref=comm_ref.at[r],
                send_sem=send_sems.at[s],
                recv_sem=recv_sems.at[r],
                device_id=(right,),
                device_id_type=pl.DeviceIdType.MESH,
            )
            rdma.start()
            rdma.wait()
            out_ref[...] += comm_ref[r, :, :].astype(jnp.float32)

    return pl.pallas_call(
        body,
        out_shape=jax.ShapeDtypeStruct((m, n), jnp.float32),
        in_specs=[
            pl.BlockSpec(memory_space=pltpu.VMEM),
            pl.BlockSpec(memory_space=pltpu.VMEM),
        ],
        out_specs=pl.BlockSpec(memory_space=pltpu.VMEM),
        scratch_shapes=[
            pltpu.VMEM((2, m, n), jnp.bfloat16),
            pltpu.SemaphoreType.DMA((2,)),
            pltpu.SemaphoreType.DMA((2,)),
        ],
        compiler_params=pltpu.CompilerParams(collective_id=0),
    )(A, B)


# baseline (device time: 315747 ns/iter reference)
import jax
import jax.numpy as jnp
from jax import lax
from jax.experimental import pallas as pl
from jax.experimental.pallas import tpu as pltpu

N_DEV = 4


def kernel(A, B):
    m, k = A.shape
    _, n = B.shape

    def body(a_ref, b_ref, out_ref, comm_ref, send_sems, recv_sems):
        my = lax.axis_index("i")
        left = lax.rem(my + N_DEV - 1, N_DEV)
        right = lax.rem(my + 1, N_DEV)

        barrier_sem = pltpu.get_barrier_semaphore()
        for nbr in (left, right):
            pl.semaphore_signal(
                barrier_sem, inc=1,
                device_id=(nbr,), device_id_type=pl.DeviceIdType.MESH,
            )
        pl.semaphore_wait(barrier_sem, 2)

        p = jnp.dot(
            a_ref[...].astype(jnp.bfloat16),
            b_ref[...].astype(jnp.bfloat16),
            preferred_element_type=jnp.float32,
        )
        out_ref[...] = p
        comm_ref[0, :, :] = p.astype(jnp.bfloat16)

        for h in range(N_DEV - 1):
            s, r = h % 2, (h + 1) % 2
            rdma = pltpu.make_async_remote_copy(
                src_ref=comm_ref.at[s],
                dst_ref=comm_ref.at[r],
                send_sem=send_sems.at[s],
                recv_sem=recv_sems.at[r],
                device_id=(right,),
                device_id_type=pl.DeviceIdType.MESH,
            )
            rdma.start()
            rdma.wait()
            out_ref[...] += comm_ref[r, :, :].astype(jnp.float32)

    return pl.pallas_call(
        body,
        out_shape=jax.ShapeDtypeStruct((m, n), jnp.float32),
        in_specs=[
            pl.BlockSpec(memory_space=pltpu.VMEM),
            pl.BlockSpec(memory_space=pltpu.VMEM),
        ],
        out_specs=pl.BlockSpec(memory_space=pltpu.VMEM),
        scratch_shapes=[
            pltpu.VMEM((2, m, n), jnp.bfloat16),
            pltpu.SemaphoreType.DMA((2,)),
            pltpu.SemaphoreType.DMA((2,)),
        ],
        compiler_params=pltpu.CompilerParams(
            collective_id=0,
            vmem_limit_bytes=100 * 1024 * 1024,
        ),
    )(A, B)


# device time: 174366 ns/iter; 1.8108x vs baseline; 1.8108x over previous
import jax
import jax.numpy as jnp
from jax import lax
from jax.experimental import pallas as pl
from jax.experimental.pallas import tpu as pltpu

N_DEV = 4


def kernel(A, B):
    m, k = A.shape
    _, n = B.shape
    mc = m // N_DEV

    def body(a_ref, b_ref, out_ref, recv_ref, send_sems, rs_sems, ag_sems):
        my = lax.axis_index("i")
        left = lax.rem(my + N_DEV - 1, N_DEV)
        right = lax.rem(my + 1, N_DEV)

        barrier_sem = pltpu.get_barrier_semaphore()
        for nbr in (left, right):
            pl.semaphore_signal(
                barrier_sem, inc=1,
                device_id=(nbr,), device_id_type=pl.DeviceIdType.MESH,
            )
        pl.semaphore_wait(barrier_sem, 2)

        p = jnp.dot(
            a_ref[...].astype(jnp.bfloat16),
            b_ref[...].astype(jnp.bfloat16),
            preferred_element_type=jnp.float32,
        )
        out_ref[...] = p.astype(jnp.bfloat16)

        for s in range(N_DEV - 1):
            sc = lax.rem(my - s + N_DEV, N_DEV)
            rc = lax.rem(my - s - 1 + N_DEV, N_DEV)
            rdma = pltpu.make_async_remote_copy(
                src_ref=out_ref.at[pl.ds(sc * mc, mc), :],
                dst_ref=recv_ref.at[s],
                send_sem=send_sems.at[s],
                recv_sem=rs_sems.at[s],
                device_id=(right,),
                device_id_type=pl.DeviceIdType.MESH,
            )
            rdma.start()
            rdma.wait()
            out_ref[pl.ds(rc * mc, mc), :] += recv_ref[s, :, :]

        for s in range(N_DEV - 1):
            sc = lax.rem(my + 1 - s + N_DEV, N_DEV)
            rdma = pltpu.make_async_remote_copy(
                src_ref=out_ref.at[pl.ds(sc * mc, mc), :],
                dst_ref=out_ref.at[pl.ds(sc * mc, mc), :],
                send_sem=send_sems.at[N_DEV - 1 + s],
                recv_sem=ag_sems.at[s],
                device_id=(right,),
                device_id_type=pl.DeviceIdType.MESH,
            )
            rdma.start()
            rdma.wait()

    return pl.pallas_call(
        body,
        out_shape=jax.ShapeDtypeStruct((m, n), jnp.bfloat16),
        in_specs=[
            pl.BlockSpec(memory_space=pltpu.VMEM),
            pl.BlockSpec(memory_space=pltpu.VMEM),
        ],
        out_specs=pl.BlockSpec(memory_space=pltpu.VMEM),
        scratch_shapes=[
            pltpu.VMEM((N_DEV - 1, mc, n), jnp.bfloat16),
            pltpu.SemaphoreType.DMA((2 * (N_DEV - 1),)),
            pltpu.SemaphoreType.DMA((N_DEV - 1,)),
            pltpu.SemaphoreType.DMA((N_DEV - 1,)),
        ],
        compiler_params=pltpu.CompilerParams(
            collective_id=0,
            vmem_limit_bytes=100 * 1024 * 1024,
        ),
    )(A, B)


# device time: 106995 ns/iter; 2.9510x vs baseline; 1.6297x over previous
import jax
import jax.numpy as jnp
from jax import lax
from jax.experimental import pallas as pl
from jax.experimental.pallas import tpu as pltpu

N_DEV = 4


def kernel(A, B):
    m, k = A.shape
    _, n = B.shape
    mc = m // N_DEV
    hw = n // 2

    def body(a_ref, b_ref, out_ref, recv_cw, recv_ccw,
             send_cw_sems, send_ccw_sems, rs_cw_sems, rs_ccw_sems,
             ag_cw_sems, ag_ccw_sems):
        my = lax.axis_index("i")
        left = lax.rem(my + N_DEV - 1, N_DEV)
        right = lax.rem(my + 1, N_DEV)

        barrier_sem = pltpu.get_barrier_semaphore()
        for nbr in (left, right):
            pl.semaphore_signal(
                barrier_sem, inc=1,
                device_id=(nbr,), device_id_type=pl.DeviceIdType.MESH,
            )
        pl.semaphore_wait(barrier_sem, 2)

        p = jnp.dot(
            a_ref[...].astype(jnp.bfloat16),
            b_ref[...].astype(jnp.bfloat16),
            preferred_element_type=jnp.float32,
        )
        out_ref[...] = p.astype(jnp.bfloat16)

        for s in range(N_DEV - 1):
            sc_cw = lax.rem(my - s + N_DEV, N_DEV)
            rc_cw = lax.rem(my - s - 1 + N_DEV, N_DEV)
            sc_ccw = lax.rem(my + s, N_DEV)
            rc_ccw = lax.rem(my + s + 1, N_DEV)
            cw = pltpu.make_async_remote_copy(
                src_ref=out_ref.at[pl.ds(sc_cw * mc, mc), pl.ds(0, hw)],
                dst_ref=recv_cw.at[s],
                send_sem=send_cw_sems.at[s],
                recv_sem=rs_cw_sems.at[s],
                device_id=(right,),
                device_id_type=pl.DeviceIdType.MESH,
            )
            ccw = pltpu.make_async_remote_copy(
                src_ref=out_ref.at[pl.ds(sc_ccw * mc, mc), pl.ds(hw, hw)],
                dst_ref=recv_ccw.at[s],
                send_sem=send_ccw_sems.at[s],
                recv_sem=rs_ccw_sems.at[s],
                device_id=(left,),
                device_id_type=pl.DeviceIdType.MESH,
            )
            cw.start()
            ccw.start()
            cw.wait()
            out_ref[pl.ds(rc_cw * mc, mc), pl.ds(0, hw)] += recv_cw[s, :, :]
            ccw.wait()
            out_ref[pl.ds(rc_ccw * mc, mc), pl.ds(hw, hw)] += recv_ccw[s, :, :]

        for s in range(N_DEV - 1):
            sc_cw = lax.rem(my + 1 - s + N_DEV, N_DEV)
            sc_ccw = lax.rem(my - 1 + s + N_DEV, N_DEV)
            cw = pltpu.make_async_remote_copy(
                src_ref=out_ref.at[pl.ds(sc_cw * mc, mc), pl.ds(0, hw)],
                dst_ref=out_ref.at[pl.ds(sc_cw * mc, mc), pl.ds(0, hw)],
                send_sem=send_cw_sems.at[N_DEV - 1 + s],
                recv_sem=ag_cw_sems.at[s],
                device_id=(right,),
                device_id_type=pl.DeviceIdType.MESH,
            )
            ccw = pltpu.make_async_remote_copy(
                src_ref=out_ref.at[pl.ds(sc_ccw * mc, mc), pl.ds(hw, hw)],
                dst_ref=out_ref.at[pl.ds(sc_ccw * mc, mc), pl.ds(hw, hw)],
                send_sem=send_ccw_sems.at[N_DEV - 1 + s],
                recv_sem=ag_ccw_sems.at[s],
                device_id=(left,),
                device_id_type=pl.DeviceIdType.MESH,
            )
            cw.start()
            ccw.start()
            cw.wait()
            ccw.wait()

    return pl.pallas_call(
        body,
        out_shape=jax.ShapeDtypeStruct((m, n), jnp.bfloat16),
        in_specs=[
            pl.BlockSpec(memory_space=pltpu.VMEM),
            pl.BlockSpec(memory_space=pltpu.VMEM),
        ],
        out_specs=pl.BlockSpec(memory_space=pltpu.VMEM),
        scratch_shapes=[
            pltpu.VMEM((N_DEV - 1, mc, hw), jnp.bfloat16),
            pltpu.VMEM((N_DEV - 1, mc, hw), jnp.bfloat16),
            pltpu.SemaphoreType.DMA((2 * (N_DEV - 1),)),
            pltpu.SemaphoreType.DMA((2 * (N_DEV - 1),)),
            pltpu.SemaphoreType.DMA((N_DEV - 1,)),
            pltpu.SemaphoreType.DMA((N_DEV - 1,)),
            pltpu.SemaphoreType.DMA((N_DEV - 1,)),
            pltpu.SemaphoreType.DMA((N_DEV - 1,)),
        ],
        compiler_params=pltpu.CompilerParams(
            collective_id=0,
            vmem_limit_bytes=100 * 1024 * 1024,
        ),
    )(A, B)


# device time: 99963 ns/iter; 3.1586x vs baseline; 1.0703x over previous
import jax
import jax.numpy as jnp
from jax import lax
from jax.experimental import pallas as pl
from jax.experimental.pallas import tpu as pltpu

N_DEV = 4


def kernel(A, B):
    m, k = A.shape
    _, n = B.shape
    mc = m // N_DEV
    hw = n // 2

    def body(a_ref, b_ref, out_ref, recv_cw, recv_ccw,
             send_cw_sems, send_ccw_sems, rs_cw_sems, rs_ccw_sems,
             ag_cw_sems, ag_ccw_sems):
        my = lax.axis_index("i")
        left = lax.rem(my + N_DEV - 1, N_DEV)
        right = lax.rem(my + 1, N_DEV)

        barrier_sem = pltpu.get_barrier_semaphore()
        for nbr in (left, right):
            pl.semaphore_signal(
                barrier_sem, inc=1,
                device_id=(nbr,), device_id_type=pl.DeviceIdType.MESH,
            )
        pl.semaphore_wait(barrier_sem, 2)

        bq = b_ref[...].astype(jnp.bfloat16)

        def compute_chunk(c):
            rows = pl.ds(c * mc, mc)
            p = jnp.dot(
                a_ref[rows, :].astype(jnp.bfloat16),
                bq,
                preferred_element_type=jnp.float32,
            )
            out_ref[rows, :] = p.astype(jnp.bfloat16)

        compute_chunk(my)

        for s in range(N_DEV - 1):
            sc_cw = lax.rem(my - s + N_DEV, N_DEV)
            rc_cw = lax.rem(my - s - 1 + N_DEV, N_DEV)
            sc_ccw = lax.rem(my + s, N_DEV)
            rc_ccw = lax.rem(my + s + 1, N_DEV)
            cw = pltpu.make_async_remote_copy(
                src_ref=out_ref.at[pl.ds(sc_cw * mc, mc), pl.ds(0, hw)],
                dst_ref=recv_cw.at[s],
                send_sem=send_cw_sems.at[s],
                recv_sem=rs_cw_sems.at[s],
                device_id=(right,),
                device_id_type=pl.DeviceIdType.MESH,
            )
            ccw = pltpu.make_async_remote_copy(
                src_ref=out_ref.at[pl.ds(sc_ccw * mc, mc), pl.ds(hw, hw)],
                dst_ref=recv_ccw.at[s],
                send_sem=send_ccw_sems.at[s],
                recv_sem=rs_ccw_sems.at[s],
                device_id=(left,),
                device_id_type=pl.DeviceIdType.MESH,
            )
            cw.start()
            ccw.start()
            if s == 0:
                compute_chunk(lax.rem(my + 1, N_DEV))
                compute_chunk(lax.rem(my + N_DEV - 1, N_DEV))
                compute_chunk(lax.rem(my + 2, N_DEV))
            cw.wait()
            out_ref[pl.ds(rc_cw * mc, mc), pl.ds(0, hw)] += recv_cw[s, :, :]
            ccw.wait()
            out_ref[pl.ds(rc_ccw * mc, mc), pl.ds(hw, hw)] += recv_ccw[s, :, :]

        for s in range(N_DEV - 1):
            sc_cw = lax.rem(my + 1 - s + N_DEV, N_DEV)
            sc_ccw = lax.rem(my - 1 + s + N_DEV, N_DEV)
            cw = pltpu.make_async_remote_copy(
                src_ref=out_ref.at[pl.ds(sc_cw * mc, mc), pl.ds(0, hw)],
                dst_ref=out_ref.at[pl.ds(sc_cw * mc, mc), pl.ds(0, hw)],
                send_sem=send_cw_sems.at[N_DEV - 1 + s],
                recv_sem=ag_cw_sems.at[s],
                device_id=(right,),
                device_id_type=pl.DeviceIdType.MESH,
            )
            ccw = pltpu.make_async_remote_copy(
                src_ref=out_ref.at[pl.ds(sc_ccw * mc, mc), pl.ds(hw, hw)],
                dst_ref=out_ref.at[pl.ds(sc_ccw * mc, mc), pl.ds(hw, hw)],
                send_sem=send_ccw_sems.at[N_DEV - 1 + s],
                recv_sem=ag_ccw_sems.at[s],
                device_id=(left,),
                device_id_type=pl.DeviceIdType.MESH,
            )
            cw.start()
            ccw.start()
            cw.wait()
            ccw.wait()

    return pl.pallas_call(
        body,
        out_shape=jax.ShapeDtypeStruct((m, n), jnp.bfloat16),
        in_specs=[
            pl.BlockSpec(memory_space=pltpu.VMEM),
            pl.BlockSpec(memory_space=pltpu.VMEM),
        ],
        out_specs=pl.BlockSpec(memory_space=pltpu.VMEM),
        scratch_shapes=[
            pltpu.VMEM((N_DEV - 1, mc, hw), jnp.bfloat16),
            pltpu.VMEM((N_DEV - 1, mc, hw), jnp.bfloat16),
            pltpu.SemaphoreType.DMA((2 * (N_DEV - 1),)),
            pltpu.SemaphoreType.DMA((2 * (N_DEV - 1),)),
            pltpu.SemaphoreType.DMA((N_DEV - 1,)),
            pltpu.SemaphoreType.DMA((N_DEV - 1,)),
            pltpu.SemaphoreType.DMA((N_DEV - 1,)),
            pltpu.SemaphoreType.DMA((N_DEV - 1,)),
        ],
        compiler_params=pltpu.CompilerParams(
            collective_id=0,
            vmem_limit_bytes=100 * 1024 * 1024,
        ),
    )(A, B)


# device time: 98846 ns/iter; 3.1943x vs baseline; 1.0113x over previous
import jax
import jax.numpy as jnp
from jax import lax
from jax.experimental import pallas as pl
from jax.experimental.pallas import tpu as pltpu

N_DEV = 4


def kernel(A, B):
    m, k = A.shape
    _, n = B.shape
    hm = m // 2
    qm = m // 4
    hw = n // 2

    def body(a_ref, b_ref, out_ref, recv_h, recv_q, send_sems, recv_sems):
        my = lax.axis_index("i")
        left = lax.rem(my + N_DEV - 1, N_DEV)
        right = lax.rem(my + 1, N_DEV)

        my_x = lax.div(my, 2)
        my_y = jnp.bitwise_xor(lax.rem(my, 2), my_x)
        xp = 3 - my
        yp = jnp.bitwise_xor(my, 1)

        barrier_sem = pltpu.get_barrier_semaphore()
        for nbr in (left, right):
            pl.semaphore_signal(
                barrier_sem, inc=1,
                device_id=(nbr,), device_id_type=pl.DeviceIdType.MESH,
            )
        pl.semaphore_wait(barrier_sem, 2)

        bq = b_ref[...].astype(jnp.bfloat16)

        def compute_quad(row_start, col0):
            rows = pl.ds(row_start, hm)
            p = jnp.dot(
                a_ref[rows, :].astype(jnp.bfloat16),
                bq[:, col0:col0 + hw],
                preferred_element_type=jnp.float32,
            )
            out_ref[rows, pl.ds(col0, hw)] = p.astype(jnp.bfloat16)

        ha_keep = my_x * hm
        ha_send = (1 - my_x) * hm
        qa_keep = my_x * hm + my_y * qm
        qa_send = my_x * hm + (1 - my_y) * qm
        hb_keep = my_y * hm
        hb_send = (1 - my_y) * hm
        qb_keep = my_y * hm + my_x * qm
        qb_send = my_y * hm + (1 - my_x) * qm

        def exchange(src_rows, nrows, col0, partner, dst_ref, sem_idx):
            return pltpu.make_async_remote_copy(
                src_ref=out_ref.at[pl.ds(src_rows, nrows), pl.ds(col0, hw)],
                dst_ref=dst_ref,
                send_sem=send_sems.at[sem_idx],
                recv_sem=recv_sems.at[sem_idx],
                device_id=(partner,),
                device_id_type=pl.DeviceIdType.MESH,
            )

        compute_quad(ha_send, 0)
        compute_quad(hb_send, hw)

        st1a = exchange(ha_send, hm, 0, xp, recv_h.at[0], 0)
        st1b = exchange(hb_send, hm, hw, yp, recv_h.at[1], 1)
        st1a.start()
        st1b.start()
        compute_quad(ha_keep, 0)
        compute_quad(hb_keep, hw)
        st1a.wait()
        out_ref[pl.ds(ha_keep, hm), pl.ds(0, hw)] += recv_h[0, :, :]
        st1b.wait()
        out_ref[pl.ds(hb_keep, hm), pl.ds(hw, hw)] += recv_h[1, :, :]

        st2a = exchange(qa_send, qm, 0, yp, recv_q.at[0], 2)
        st2b = exchange(qb_send, qm, hw, xp, recv_q.at[1], 3)
        st2a.start()
        st2b.start()
        st2a.wait()
        out_ref[pl.ds(qa_keep, qm), pl.ds(0, hw)] += recv_q[0, :, :]
        st2b.wait()
        out_ref[pl.ds(qb_keep, qm), pl.ds(hw, hw)] += recv_q[1, :, :]

        st3a = exchange(
            qa_keep, qm, 0, yp,
            out_ref.at[pl.ds(qa_keep, qm), pl.ds(0, hw)], 4,
        )
        st3b = exchange(
            qb_keep, qm, hw, xp,
            out_ref.at[pl.ds(qb_keep, qm), pl.ds(hw, hw)], 5,
        )
        st3a.start()
        st3b.start()
        st3a.wait()
        st3b.wait()

        st4a = exchange(
            ha_keep, hm, 0, xp,
            out_ref.at[pl.ds(ha_keep, hm), pl.ds(0, hw)], 6,
        )
        st4b = exchange(
            hb_keep, hm, hw, yp,
            out_ref.at[pl.ds(hb_keep, hm), pl.ds(hw, hw)], 7,
        )
        st4a.start()
        st4b.start()
        st4a.wait()
        st4b.wait()

    return pl.pallas_call(
        body,
        out_shape=jax.ShapeDtypeStruct((m, n), jnp.bfloat16),
        in_specs=[
            pl.BlockSpec(memory_space=pltpu.VMEM),
            pl.BlockSpec(memory_space=pltpu.VMEM),
        ],
        out_specs=pl.BlockSpec(memory_space=pltpu.VMEM),
        scratch_shapes=[
            pltpu.VMEM((2, hm, hw), jnp.bfloat16),
            pltpu.VMEM((2, qm, hw), jnp.bfloat16),
            pltpu.SemaphoreType.DMA((8,)),
            pltpu.SemaphoreType.DMA((8,)),
        ],
        compiler_params=pltpu.CompilerParams(
            collective_id=0,
            vmem_limit_bytes=100 * 1024 * 1024,
        ),
    )(A, B)


# device time: 98641 ns/iter; 3.2010x vs baseline; 1.0021x over previous
import jax
import jax.numpy as jnp
from jax import lax
from jax.experimental import pallas as pl
from jax.experimental.pallas import tpu as pltpu

N_DEV = 4


def kernel(A, B):
    m, k = A.shape
    _, n = B.shape
    hm = m // 2
    qm = m // 4
    hw = n // 2

    def body(a_ref, b_ref, out_ref, recv_h, recv_q, send_sems, recv_sems):
        my = lax.axis_index("i")
        left = lax.rem(my + N_DEV - 1, N_DEV)
        right = lax.rem(my + 1, N_DEV)

        my_x = lax.div(my, 2)
        my_y = jnp.bitwise_xor(lax.rem(my, 2), my_x)
        xp = 3 - my
        yp = jnp.bitwise_xor(my, 1)

        barrier_sem = pltpu.get_barrier_semaphore()
        for nbr in (left, right):
            pl.semaphore_signal(
                barrier_sem, inc=1,
                device_id=(nbr,), device_id_type=pl.DeviceIdType.MESH,
            )
        pl.semaphore_wait(barrier_sem, 2)

        bq = b_ref[...].astype(jnp.bfloat16)

        def compute_quad(row_start, col0):
            rows = pl.ds(row_start, hm)
            p = jnp.dot(
                a_ref[rows, :].astype(jnp.bfloat16),
                bq[:, col0:col0 + hw],
                preferred_element_type=jnp.float32,
            )
            out_ref[rows, pl.ds(col0, hw)] = p.astype(jnp.bfloat16)

        ha_keep = my_x * hm
        ha_send = (1 - my_x) * hm
        qa_keep = my_x * hm + my_y * qm
        qa_send = my_x * hm + (1 - my_y) * qm
        hb_keep = my_y * hm
        hb_send = (1 - my_y) * hm
        qb_keep = my_y * hm + my_x * qm
        qb_send = my_y * hm + (1 - my_x) * qm

        def exchange(src_rows, nrows, col0, partner, dst_ref, sem_idx):
            return pltpu.make_async_remote_copy(
                src_ref=out_ref.at[pl.ds(src_rows, nrows), pl.ds(col0, hw)],
                dst_ref=dst_ref,
                send_sem=send_sems.at[sem_idx],
                recv_sem=recv_sems.at[sem_idx],
                device_id=(partner,),
                device_id_type=pl.DeviceIdType.MESH,
            )

        compute_quad(ha_send, 0)
        compute_quad(hb_send, hw)

        st1a = exchange(ha_send, hm, 0, xp, recv_h.at[0], 0)
        st1b = exchange(hb_send, hm, hw, yp, recv_h.at[1], 1)
        st1a.start()
        st1b.start()
        compute_quad(ha_keep, 0)
        compute_quad(hb_keep, hw)
        oa_send = (1 - my_y) * qm
        oa_keep = my_y * qm
        ob_send = (1 - my_x) * qm
        ob_keep = my_x * qm

        st2a = exchange(qa_send, qm, 0, yp, recv_q.at[0], 2)
        st2b = exchange(qb_send, qm, hw, xp, recv_q.at[1], 3)

        st1a.wait()
        out_ref[pl.ds(qa_send, qm), pl.ds(0, hw)] += recv_h[0, pl.ds(oa_send, qm), :]
        st2a.start()
        out_ref[pl.ds(qa_keep, qm), pl.ds(0, hw)] += recv_h[0, pl.ds(oa_keep, qm), :]
        st1b.wait()
        out_ref[pl.ds(qb_send, qm), pl.ds(hw, hw)] += recv_h[1, pl.ds(ob_send, qm), :]
        st2b.start()
        out_ref[pl.ds(qb_keep, qm), pl.ds(hw, hw)] += recv_h[1, pl.ds(ob_keep, qm), :]

        st3a = exchange(
            qa_keep, qm, 0, yp,
            out_ref.at[pl.ds(qa_keep, qm), pl.ds(0, hw)], 4,
        )
        st3b = exchange(
            qb_keep, qm, hw, xp,
            out_ref.at[pl.ds(qb_keep, qm), pl.ds(hw, hw)], 5,
        )
        st2a.wait()
        out_ref[pl.ds(qa_keep, qm), pl.ds(0, hw)] += recv_q[0, :, :]
        st3a.start()
        st2b.wait()
        out_ref[pl.ds(qb_keep, qm), pl.ds(hw, hw)] += recv_q[1, :, :]
        st3b.start()

        st4a = exchange(
            ha_keep, hm, 0, xp,
            out_ref.at[pl.ds(ha_keep, hm), pl.ds(0, hw)], 6,
        )
        st4b = exchange(
            hb_keep, hm, hw, yp,
            out_ref.at[pl.ds(hb_keep, hm), pl.ds(hw, hw)], 7,
        )
        st3a.wait()
        st4a.start()
        st3b.wait()
        st4b.start()
        st4a.wait()
        st4b.wait()

    return pl.pallas_call(
        body,
        out_shape=jax.ShapeDtypeStruct((m, n), jnp.bfloat16),
        in_specs=[
            pl.BlockSpec(memory_space=pltpu.VMEM),
            pl.BlockSpec(memory_space=pltpu.VMEM),
        ],
        out_specs=pl.BlockSpec(memory_space=pltpu.VMEM),
        scratch_shapes=[
            pltpu.VMEM((2, hm, hw), jnp.bfloat16),
            pltpu.VMEM((2, qm, hw), jnp.bfloat16),
            pltpu.SemaphoreType.DMA((8,)),
            pltpu.SemaphoreType.DMA((8,)),
        ],
        compiler_params=pltpu.CompilerParams(
            collective_id=0,
            vmem_limit_bytes=100 * 1024 * 1024,
        ),
    )(A, B)


# device time: 93155 ns/iter; 3.3895x vs baseline; 1.0589x over previous
import jax
import jax.numpy as jnp
from jax import lax
from jax.experimental import pallas as pl
from jax.experimental.pallas import tpu as pltpu

N_DEV = 4


def kernel(A, B):
    m, k = A.shape
    _, n = B.shape
    hm = m // 2
    qm = m // 4
    hq = m // 8
    hw = n // 2

    def body(a_ref, b_ref, out_ref, recv_h, recv_q, send_sems, recv_sems):
        my = lax.axis_index("i")
        left = lax.rem(my + N_DEV - 1, N_DEV)
        right = lax.rem(my + 1, N_DEV)

        my_x = lax.div(my, 2)
        my_y = jnp.bitwise_xor(lax.rem(my, 2), my_x)
        xp = 3 - my
        yp = jnp.bitwise_xor(my, 1)

        barrier_sem = pltpu.get_barrier_semaphore()
        for nbr in (left, right):
            pl.semaphore_signal(
                barrier_sem, inc=1,
                device_id=(nbr,), device_id_type=pl.DeviceIdType.MESH,
            )
        pl.semaphore_wait(barrier_sem, 2)

        bq = b_ref[...].astype(jnp.bfloat16)

        def compute_block(row0, nrows, col0):
            rows = pl.ds(row0, nrows)
            p = jnp.dot(
                a_ref[rows, :].astype(jnp.bfloat16),
                bq[:, col0:col0 + hw],
                preferred_element_type=jnp.float32,
            )
            out_ref[rows, pl.ds(col0, hw)] = p.astype(jnp.bfloat16)

        ha_keep = my_x * hm
        ha_send = (1 - my_x) * hm
        qa_keep = my_x * hm + my_y * qm
        qa_send = my_x * hm + (1 - my_y) * qm
        hb_keep = my_y * hm
        hb_send = (1 - my_y) * hm
        qb_keep = my_y * hm + my_x * qm
        qb_send = my_y * hm + (1 - my_x) * qm
        oa_send = (1 - my_y) * qm
        oa_keep = my_y * qm
        ob_send = (1 - my_x) * qm
        ob_keep = my_x * qm

        def ex(row0, nrows, col0, partner, dst_ref, idx):
            return pltpu.make_async_remote_copy(
                src_ref=out_ref.at[pl.ds(row0, nrows), pl.ds(col0, hw)],
                dst_ref=dst_ref,
                send_sem=send_sems.at[idx],
                recv_sem=recv_sems.at[idx],
                device_id=(partner,),
                device_id_type=pl.DeviceIdType.MESH,
            )

        compute_block(ha_send + oa_send, qm, 0)
        a1 = ex(ha_send + oa_send, qm, 0, xp,
                recv_h.at[0, pl.ds(oa_send, qm), :], 0)
        a1.start()
        compute_block(hb_send + ob_send, qm, hw)
        b1 = ex(hb_send + ob_send, qm, hw, yp,
                recv_h.at[1, pl.ds(ob_send, qm), :], 10)
        b1.start()
        compute_block(ha_send + oa_keep, qm, 0)
        a2 = ex(ha_send + oa_keep, qm, 0, xp,
                recv_h.at[0, pl.ds(oa_keep, qm), :], 1)
        a2.start()
        compute_block(hb_send + ob_keep, qm, hw)
        b2 = ex(hb_send + ob_keep, qm, hw, yp,
                recv_h.at[1, pl.ds(ob_keep, qm), :], 11)
        b2.start()

        compute_block(qa_send, qm, 0)
        compute_block(qb_send, qm, hw)
        compute_block(qa_keep, qm, 0)
        compute_block(qb_keep, qm, hw)

        a1.wait()
        out_ref[pl.ds(qa_send, qm), pl.ds(0, hw)] += \
            recv_h[0, pl.ds(oa_send, qm), :]
        st2a = []
        for r in range(2):
            e = ex(qa_send + r * hq, hq, 0, yp,
                   recv_q.at[0, pl.ds(r * hq, hq), :], 2 + r)
            e.start()
            st2a.append(e)
        b1.wait()
        out_ref[pl.ds(qb_send, qm), pl.ds(hw, hw)] += \
            recv_h[1, pl.ds(ob_send, qm), :]
        st2b = []
        for r in range(2):
            e = ex(qb_send + r * hq, hq, hw, xp,
                   recv_q.at[1, pl.ds(r * hq, hq), :], 12 + r)
            e.start()
            st2b.append(e)
        a2.wait()
        out_ref[pl.ds(qa_keep, qm), pl.ds(0, hw)] += \
            recv_h[0, pl.ds(oa_keep, qm), :]
        b2.wait()
        out_ref[pl.ds(qb_keep, qm), pl.ds(hw, hw)] += \
            recv_h[1, pl.ds(ob_keep, qm), :]

        st3a = []
        for r in range(2):
            st2a[r].wait()
            out_ref[pl.ds(qa_keep + r * hq, hq), pl.ds(0, hw)] += \
                recv_q[0, pl.ds(r * hq, hq), :]
            e = ex(qa_keep + r * hq, hq, 0, yp,
                   out_ref.at[pl.ds(qa_keep + r * hq, hq), pl.ds(0, hw)],
                   4 + r)
            e.start()
            st3a.append(e)
        st4a1 = ex(qa_keep, qm, 0, xp,
                   out_ref.at[pl.ds(qa_keep, qm), pl.ds(0, hw)], 6)
        st4a1.start()
        st3b = []
        for r in range(2):
            st2b[r].wait()
            out_ref[pl.ds(qb_keep + r * hq, hq), pl.ds(hw, hw)] += \
                recv_q[1, pl.ds(r * hq, hq), :]
            e = ex(qb_keep + r * hq, hq, hw, xp,
                   out_ref.at[pl.ds(qb_keep + r * hq, hq), pl.ds(hw, hw)],
                   14 + r)
            e.start()
            st3b.append(e)
        st4b1 = ex(qb_keep, qm, hw, yp,
                   out_ref.at[pl.ds(qb_keep, qm), pl.ds(hw, hw)], 16)
        st4b1.start()

        st4a2 = []
        for r in range(2):
            st3a[r].wait()
            e = ex(qa_send + r * hq, hq, 0, xp,
                   out_ref.at[pl.ds(qa_send + r * hq, hq), pl.ds(0, hw)],
                   7 + r)
            e.start()
            st4a2.append(e)
        st4b2 = []
        for r in range(2):
            st3b[r].wait()
            e = ex(qb_send + r * hq, hq, hw, yp,
                   out_ref.at[pl.ds(qb_send + r * hq, hq), pl.ds(hw, hw)],
                   17 + r)
            e.start()
            st4b2.append(e)

        st4a1.wait()
        st4b1.wait()
        for e in st4a2 + st4b2:
            e.wait()

    return pl.pallas_call(
        body,
        out_shape=jax.ShapeDtypeStruct((m, n), jnp.bfloat16),
        in_specs=[
            pl.BlockSpec(memory_space=pltpu.VMEM),
            pl.BlockSpec(memory_space=pltpu.VMEM),
        ],
        out_specs=pl.BlockSpec(memory_space=pltpu.VMEM),
        scratch_shapes=[
            pltpu.VMEM((2, hm, hw), jnp.bfloat16),
            pltpu.VMEM((2, qm, hw), jnp.bfloat16),
            pltpu.SemaphoreType.DMA((20,)),
            pltpu.SemaphoreType.DMA((20,)),
        ],
        compiler_params=pltpu.CompilerParams(
            collective_id=0,
            vmem_limit_bytes=100 * 1024 * 1024,
        ),
    )(A, B)


# device time: 91559 ns/iter; 3.4486x vs baseline; 1.0174x over previous
import jax
import jax.numpy as jnp
from jax import lax
from jax.experimental import pallas as pl
from jax.experimental.pallas import tpu as pltpu

N_DEV = 4


def kernel(A, B):
    m, k = A.shape
    _, n = B.shape
    hm = m // 2
    qm = m // 4
    hq = m // 8
    hw = n // 2

    def body(a_hbm, b_hbm, out_ref, a_ref, b_ref, in_sems, recv_h, recv_q, send_sems, recv_sems):
        my = lax.axis_index("i")
        left = lax.rem(my + N_DEV - 1, N_DEV)
        right = lax.rem(my + 1, N_DEV)

        my_x = lax.div(my, 2)
        my_y = jnp.bitwise_xor(lax.rem(my, 2), my_x)
        xp = 3 - my
        yp = jnp.bitwise_xor(my, 1)

        ha_send_e = (1 - my_x) * hm
        ha_keep_e = my_x * hm

        dma_b_low = pltpu.make_async_copy(
            b_hbm.at[:, pl.ds(0, hw)], b_ref.at[:, pl.ds(0, hw)],
            in_sems.at[0])
        dma_a_1 = pltpu.make_async_copy(
            a_hbm.at[pl.ds(ha_send_e, hm), :],
            a_ref.at[pl.ds(ha_send_e, hm), :], in_sems.at[1])
        dma_b_high = pltpu.make_async_copy(
            b_hbm.at[:, pl.ds(hw, hw)], b_ref.at[:, pl.ds(hw, hw)],
            in_sems.at[2])
        dma_a_2 = pltpu.make_async_copy(
            a_hbm.at[pl.ds(ha_keep_e, hm), :],
            a_ref.at[pl.ds(ha_keep_e, hm), :], in_sems.at[3])
        dma_b_low.start()
        dma_a_1.start()
        dma_b_high.start()
        dma_a_2.start()

        barrier_sem = pltpu.get_barrier_semaphore()
        for nbr in (left, right):
            pl.semaphore_signal(
                barrier_sem, inc=1,
                device_id=(nbr,), device_id_type=pl.DeviceIdType.MESH,
            )
        pl.semaphore_wait(barrier_sem, 2)

        def make_compute(col0):
            bqh = b_ref[:, pl.ds(col0, hw)].astype(jnp.bfloat16)

            def compute_block(row0, nrows):
                rows = pl.ds(row0, nrows)
                p = jnp.dot(
                    a_ref[rows, :].astype(jnp.bfloat16),
                    bqh,
                    preferred_element_type=jnp.float32,
                )
                out_ref[rows, pl.ds(col0, hw)] = p.astype(jnp.bfloat16)

            return compute_block

        ha_keep = my_x * hm
        ha_send = (1 - my_x) * hm
        qa_keep = my_x * hm + my_y * qm
        qa_send = my_x * hm + (1 - my_y) * qm
        hb_keep = my_y * hm
        hb_send = (1 - my_y) * hm
        qb_keep = my_y * hm + my_x * qm
        qb_send = my_y * hm + (1 - my_x) * qm
        oa_send = (1 - my_y) * qm
        oa_keep = my_y * qm
        ob_send = (1 - my_x) * qm
        ob_keep = my_x * qm

        def ex(row0, nrows, col0, partner, dst_ref, idx):
            return pltpu.make_async_remote_copy(
                src_ref=out_ref.at[pl.ds(row0, nrows), pl.ds(col0, hw)],
                dst_ref=dst_ref,
                send_sem=send_sems.at[idx],
                recv_sem=recv_sems.at[idx],
                device_id=(partner,),
                device_id_type=pl.DeviceIdType.MESH,
            )

        dma_b_low.wait()
        dma_a_1.wait()
        compute_low = make_compute(0)
        compute_low(ha_send + oa_send, qm)
        a1 = ex(ha_send + oa_send, qm, 0, xp,
                recv_h.at[0, pl.ds(oa_send, qm), :], 0)
        a1.start()
        dma_b_high.wait()
        dma_a_2.wait()
        compute_high = make_compute(hw)
        compute_high(hb_send + ob_send, qm)
        b1 = ex(hb_send + ob_send, qm, hw, yp,
                recv_h.at[1, pl.ds(ob_send, qm), :], 10)
        b1.start()
        compute_low(ha_send + oa_keep, qm)
        a2 = ex(ha_send + oa_keep, qm, 0, xp,
                recv_h.at[0, pl.ds(oa_keep, qm), :], 1)
        a2.start()
        compute_high(hb_send + ob_keep, qm)
        b2 = ex(hb_send + ob_keep, qm, hw, yp,
                recv_h.at[1, pl.ds(ob_keep, qm), :], 11)
        b2.start()

        compute_low(qa_send, qm)
        compute_high(qb_send, qm)
        compute_low(qa_keep, qm)
        compute_high(qb_keep, qm)

        a1.wait()
        out_ref[pl.ds(qa_send, qm), pl.ds(0, hw)] += \
            recv_h[0, pl.ds(oa_send, qm), :]
        st2a = []
        for r in range(2):
            e = ex(qa_send + r * hq, hq, 0, yp,
                   recv_q.at[0, pl.ds(r * hq, hq), :], 2 + r)
            e.start()
            st2a.append(e)
        b1.wait()
        out_ref[pl.ds(qb_send, qm), pl.ds(hw, hw)] += \
            recv_h[1, pl.ds(ob_send, qm), :]
        st2b = []
        for r in range(2):
            e = ex(qb_send + r * hq, hq, hw, xp,
                   recv_q.at[1, pl.ds(r * hq, hq), :], 12 + r)
            e.start()
            st2b.append(e)
        a2.wait()
        out_ref[pl.ds(qa_keep, qm), pl.ds(0, hw)] += \
            recv_h[0, pl.ds(oa_keep, qm), :]
        b2.wait()
        out_ref[pl.ds(qb_keep, qm), pl.ds(hw, hw)] += \
            recv_h[1, pl.ds(ob_keep, qm), :]

        st3a = []
        for r in range(2):
            st2a[r].wait()
            out_ref[pl.ds(qa_keep + r * hq, hq), pl.ds(0, hw)] += \
                recv_q[0, pl.ds(r * hq, hq), :]
            e = ex(qa_keep + r * hq, hq, 0, yp,
                   out_ref.at[pl.ds(qa_keep + r * hq, hq), pl.ds(0, hw)],
                   4 + r)
            e.start()
            st3a.append(e)
        st4a1 = ex(qa_keep, qm, 0, xp,
                   out_ref.at[pl.ds(qa_keep, qm), pl.ds(0, hw)], 6)
        st4a1.start()
        st3b = []
        for r in range(2):
            st2b[r].wait()
            out_ref[pl.ds(qb_keep + r * hq, hq), pl.ds(hw, hw)] += \
                recv_q[1, pl.ds(r * hq, hq), :]
            e = ex(qb_keep + r * hq, hq, hw, xp,
                   out_ref.at[pl.ds(qb_keep + r * hq, hq), pl.ds(hw, hw)],
                   14 + r)
            e.start()
            st3b.append(e)
        st4b1 = ex(qb_keep, qm, hw, yp,
                   out_ref.at[pl.ds(qb_keep, qm), pl.ds(hw, hw)], 16)
        st4b1.start()

        st4a2 = []
        for r in range(2):
            st3a[r].wait()
            e = ex(qa_send + r * hq, hq, 0, xp,
                   out_ref.at[pl.ds(qa_send + r * hq, hq), pl.ds(0, hw)],
                   7 + r)
            e.start()
            st4a2.append(e)
        st4b2 = []
        for r in range(2):
            st3b[r].wait()
            e = ex(qb_send + r * hq, hq, hw, yp,
                   out_ref.at[pl.ds(qb_send + r * hq, hq), pl.ds(hw, hw)],
                   17 + r)
            e.start()
            st4b2.append(e)

        st4a1.wait()
        st4b1.wait()
        for e in st4a2 + st4b2:
            e.wait()

    return pl.pallas_call(
        body,
        out_shape=jax.ShapeDtypeStruct((m, n), jnp.bfloat16),
        in_specs=[
            pl.BlockSpec(memory_space=pltpu.MemorySpace.HBM),
            pl.BlockSpec(memory_space=pltpu.MemorySpace.HBM),
        ],
        out_specs=pl.BlockSpec(memory_space=pltpu.VMEM),
        scratch_shapes=[
            pltpu.VMEM((m, k), jnp.float32),
            pltpu.VMEM((k, n), jnp.float32),
            pltpu.SemaphoreType.DMA((4,)),
            pltpu.VMEM((2, hm, hw), jnp.bfloat16),
            pltpu.VMEM((2, qm, hw), jnp.bfloat16),
            pltpu.SemaphoreType.DMA((20,)),
            pltpu.SemaphoreType.DMA((20,)),
        ],
        compiler_params=pltpu.CompilerParams(
            collective_id=0,
            vmem_limit_bytes=100 * 1024 * 1024,
        ),
    )(A, B)


# device time: 88290 ns/iter; 3.5762x vs baseline; 1.0370x over previous
import jax
import jax.numpy as jnp
from jax import lax
from jax.experimental import pallas as pl
from jax.experimental.pallas import tpu as pltpu

N_DEV = 4


def kernel(A, B):
    m, k = A.shape
    _, n = B.shape
    hm = m // 2
    qm = m // 4
    hq = m // 8
    hw = n // 2

    def body(a_hbm, b_hbm, out_hbm, a_ref, b_ref, in_sems, out_ref, out_sems, recv_h, recv_q, send_sems, recv_sems):
        my = lax.axis_index("i")
        left = lax.rem(my + N_DEV - 1, N_DEV)
        right = lax.rem(my + 1, N_DEV)

        my_x = lax.div(my, 2)
        my_y = jnp.bitwise_xor(lax.rem(my, 2), my_x)
        xp = 3 - my
        yp = jnp.bitwise_xor(my, 1)

        ha_send_e = (1 - my_x) * hm
        ha_keep_e = my_x * hm

        dma_b_low = pltpu.make_async_copy(
            b_hbm.at[:, pl.ds(0, hw)], b_ref.at[:, pl.ds(0, hw)],
            in_sems.at[0])
        dma_a_1 = pltpu.make_async_copy(
            a_hbm.at[pl.ds(ha_send_e, hm), :],
            a_ref.at[pl.ds(ha_send_e, hm), :], in_sems.at[1])
        dma_b_high = pltpu.make_async_copy(
            b_hbm.at[:, pl.ds(hw, hw)], b_ref.at[:, pl.ds(hw, hw)],
            in_sems.at[2])
        dma_a_2 = pltpu.make_async_copy(
            a_hbm.at[pl.ds(ha_keep_e, hm), :],
            a_ref.at[pl.ds(ha_keep_e, hm), :], in_sems.at[3])
        dma_b_low.start()
        dma_a_1.start()
        dma_b_high.start()
        dma_a_2.start()

        barrier_sem = pltpu.get_barrier_semaphore()
        for nbr in (left, right):
            pl.semaphore_signal(
                barrier_sem, inc=1,
                device_id=(nbr,), device_id_type=pl.DeviceIdType.MESH,
            )
        pl.semaphore_wait(barrier_sem, 2)

        def make_compute(col0):
            bqh = b_ref[:, pl.ds(col0, hw)].astype(jnp.bfloat16)

            def compute_block(row0, nrows):
                rows = pl.ds(row0, nrows)
                p = jnp.dot(
                    a_ref[rows, :].astype(jnp.bfloat16),
                    bqh,
                    preferred_element_type=jnp.float32,
                )
                out_ref[rows, pl.ds(col0, hw)] = p.astype(jnp.bfloat16)

            return compute_block

        ha_keep = my_x * hm
        ha_send = (1 - my_x) * hm
        qa_keep = my_x * hm + my_y * qm
        qa_send = my_x * hm + (1 - my_y) * qm
        hb_keep = my_y * hm
        hb_send = (1 - my_y) * hm
        qb_keep = my_y * hm + my_x * qm
        qb_send = my_y * hm + (1 - my_x) * qm
        oa_send = (1 - my_y) * qm
        oa_keep = my_y * qm
        ob_send = (1 - my_x) * qm
        ob_keep = my_x * qm

        def ex(row0, nrows, col0, partner, dst_ref, idx):
            return pltpu.make_async_remote_copy(
                src_ref=out_ref.at[pl.ds(row0, nrows), pl.ds(col0, hw)],
                dst_ref=dst_ref,
                send_sem=send_sems.at[idx],
                recv_sem=recv_sems.at[idx],
                device_id=(partner,),
                device_id_type=pl.DeviceIdType.MESH,
            )

        dma_b_low.wait()
        dma_a_1.wait()
        compute_low = make_compute(0)
        compute_low(ha_send + oa_send, qm)
        a1 = ex(ha_send + oa_send, qm, 0, xp,
                recv_h.at[0, pl.ds(oa_send, qm), :], 0)
        a1.start()
        dma_b_high.wait()
        dma_a_2.wait()
        compute_high = make_compute(hw)
        compute_high(hb_send + ob_send, qm)
        b1 = ex(hb_send + ob_send, qm, hw, yp,
                recv_h.at[1, pl.ds(ob_send, qm), :], 10)
        b1.start()
        compute_low(ha_send + oa_keep, qm)
        a2 = ex(ha_send + oa_keep, qm, 0, xp,
                recv_h.at[0, pl.ds(oa_keep, qm), :], 1)
        a2.start()
        compute_high(hb_send + ob_keep, qm)
        b2 = ex(hb_send + ob_keep, qm, hw, yp,
                recv_h.at[1, pl.ds(ob_keep, qm), :], 11)
        b2.start()

        compute_low(qa_send, qm)
        compute_high(qb_send, qm)
        compute_low(qa_keep, qm)
        compute_high(qb_keep, qm)

        a1.wait()
        out_ref[pl.ds(qa_send, qm), pl.ds(0, hw)] += \
            recv_h[0, pl.ds(oa_send, qm), :]
        st2a = []
        for r in range(2):
            e = ex(qa_send + r * hq, hq, 0, yp,
                   recv_q.at[0, pl.ds(r * hq, hq), :], 2 + r)
            e.start()
            st2a.append(e)
        b1.wait()
        out_ref[pl.ds(qb_send, qm), pl.ds(hw, hw)] += \
            recv_h[1, pl.ds(ob_send, qm), :]
        st2b = []
        for r in range(2):
            e = ex(qb_send + r * hq, hq, hw, xp,
                   recv_q.at[1, pl.ds(r * hq, hq), :], 12 + r)
            e.start()
            st2b.append(e)
        a2.wait()
        out_ref[pl.ds(qa_keep, qm), pl.ds(0, hw)] += \
            recv_h[0, pl.ds(oa_keep, qm), :]
        b2.wait()
        out_ref[pl.ds(qb_keep, qm), pl.ds(hw, hw)] += \
            recv_h[1, pl.ds(ob_keep, qm), :]

        st3a = []
        for r in range(2):
            st2a[r].wait()
            out_ref[pl.ds(qa_keep + r * hq, hq), pl.ds(0, hw)] += \
                recv_q[0, pl.ds(r * hq, hq), :]
            e = ex(qa_keep + r * hq, hq, 0, yp,
                   out_ref.at[pl.ds(qa_keep + r * hq, hq), pl.ds(0, hw)],
                   4 + r)
            e.start()
            st3a.append(e)
        st4a1 = ex(qa_keep, qm, 0, xp,
                   out_hbm.at[pl.ds(qa_keep, qm), pl.ds(0, hw)], 6)
        st4a1.start()
        st3b = []
        for r in range(2):
            st2b[r].wait()
            out_ref[pl.ds(qb_keep + r * hq, hq), pl.ds(hw, hw)] += \
                recv_q[1, pl.ds(r * hq, hq), :]
            e = ex(qb_keep + r * hq, hq, hw, xp,
                   out_ref.at[pl.ds(qb_keep + r * hq, hq), pl.ds(hw, hw)],
                   14 + r)
            e.start()
            st3b.append(e)
        st4b1 = ex(qb_keep, qm, hw, yp,
                   out_hbm.at[pl.ds(qb_keep, qm), pl.ds(hw, hw)], 16)
        st4b1.start()

        st4a2 = []
        for r in range(2):
            st3a[r].wait()
            e = ex(qa_send + r * hq, hq, 0, xp,
                   out_hbm.at[pl.ds(qa_send + r * hq, hq), pl.ds(0, hw)],
                   7 + r)
            e.start()
            st4a2.append(e)
        dma_out_a = pltpu.make_async_copy(
            out_ref.at[pl.ds(ha_keep, hm), pl.ds(0, hw)],
            out_hbm.at[pl.ds(ha_keep, hm), pl.ds(0, hw)], out_sems.at[0])
        dma_out_a.start()
        st4b2 = []
        for r in range(2):
            st3b[r].wait()
            e = ex(qb_send + r * hq, hq, hw, yp,
                   out_hbm.at[pl.ds(qb_send + r * hq, hq), pl.ds(hw, hw)],
                   17 + r)
            e.start()
            st4b2.append(e)

        dma_out_b = pltpu.make_async_copy(
            out_ref.at[pl.ds(hb_keep, hm), pl.ds(hw, hw)],
            out_hbm.at[pl.ds(hb_keep, hm), pl.ds(hw, hw)], out_sems.at[1])
        dma_out_b.start()

        dma_out_a.wait()
        dma_out_b.wait()
        st4a1.wait()
        st4b1.wait()
        for e in st4a2 + st4b2:
            e.wait()

    return pl.pallas_call(
        body,
        out_shape=jax.ShapeDtypeStruct((m, n), jnp.bfloat16),
        in_specs=[
            pl.BlockSpec(memory_space=pltpu.MemorySpace.HBM),
            pl.BlockSpec(memory_space=pltpu.MemorySpace.HBM),
        ],
        out_specs=pl.BlockSpec(memory_space=pltpu.MemorySpace.HBM),
        scratch_shapes=[
            pltpu.VMEM((m, k), jnp.float32),
            pltpu.VMEM((k, n), jnp.float32),
            pltpu.SemaphoreType.DMA((4,)),
            pltpu.VMEM((m, n), jnp.bfloat16),
            pltpu.SemaphoreType.DMA((2,)),
            pltpu.VMEM((2, hm, hw), jnp.bfloat16),
            pltpu.VMEM((2, qm, hw), jnp.bfloat16),
            pltpu.SemaphoreType.DMA((20,)),
            pltpu.SemaphoreType.DMA((20,)),
        ],
        compiler_params=pltpu.CompilerParams(
            collective_id=0,
            vmem_limit_bytes=100 * 1024 * 1024,
        ),
    )(A, B)


# device time: 87740 ns/iter; 3.5987x vs baseline; 1.0063x over previous
import jax
import jax.numpy as jnp
from jax import lax
from jax.experimental import pallas as pl
from jax.experimental.pallas import tpu as pltpu

N_DEV = 4


def kernel(A, B):
    m, k = A.shape
    _, n = B.shape
    hm = m // 2
    qm = m // 4
    hq = m // 8
    hw = n // 2

    def body(a_hbm, b_hbm, out_hbm, a_ref, b_ref, in_sems, out_ref, out_sems, recv_h, recv_q, send_sems, recv_sems):
        my = lax.axis_index("i")
        left = lax.rem(my + N_DEV - 1, N_DEV)
        right = lax.rem(my + 1, N_DEV)

        my_x = lax.div(my, 2)
        my_y = jnp.bitwise_xor(lax.rem(my, 2), my_x)
        xp = 3 - my
        yp = jnp.bitwise_xor(my, 1)

        ha_send_e = (1 - my_x) * hm
        ha_keep_e = my_x * hm

        dma_b_low = pltpu.make_async_copy(
            b_hbm.at[:, pl.ds(0, hw)], b_ref.at[:, pl.ds(0, hw)],
            in_sems.at[0])
        dma_a_1 = pltpu.make_async_copy(
            a_hbm.at[pl.ds(ha_send_e, hm), :],
            a_ref.at[pl.ds(ha_send_e, hm), :], in_sems.at[1])
        dma_b_high = pltpu.make_async_copy(
            b_hbm.at[:, pl.ds(hw, hw)], b_ref.at[:, pl.ds(hw, hw)],
            in_sems.at[2])
        dma_a_2 = pltpu.make_async_copy(
            a_hbm.at[pl.ds(ha_keep_e, hm), :],
            a_ref.at[pl.ds(ha_keep_e, hm), :], in_sems.at[3])
        dma_b_low.start()
        dma_a_1.start()
        dma_b_high.start()
        dma_a_2.start()

        barrier_sem = pltpu.get_barrier_semaphore()
        for nbr in (left, right):
            pl.semaphore_signal(
                barrier_sem, inc=1,
                device_id=(nbr,), device_id_type=pl.DeviceIdType.MESH,
            )

        def make_compute(col0):
            bqh = b_ref[:, pl.ds(col0, hw)].astype(jnp.bfloat16)

            def compute_block(row0, nrows):
                rows = pl.ds(row0, nrows)
                p = jnp.dot(
                    a_ref[rows, :].astype(jnp.bfloat16),
                    bqh,
                    preferred_element_type=jnp.float32,
                )
                out_ref[rows, pl.ds(col0, hw)] = p.astype(jnp.bfloat16)

            return compute_block

        ha_keep = my_x * hm
        ha_send = (1 - my_x) * hm
        qa_keep = my_x * hm + my_y * qm
        qa_send = my_x * hm + (1 - my_y) * qm
        hb_keep = my_y * hm
        hb_send = (1 - my_y) * hm
        qb_keep = my_y * hm + my_x * qm
        qb_send = my_y * hm + (1 - my_x) * qm
        oa_send = (1 - my_y) * qm
        oa_keep = my_y * qm
        ob_send = (1 - my_x) * qm
        ob_keep = my_x * qm

        def ex(row0, nrows, col0, partner, dst_ref, idx):
            return pltpu.make_async_remote_copy(
                src_ref=out_ref.at[pl.ds(row0, nrows), pl.ds(col0, hw)],
                dst_ref=dst_ref,
                send_sem=send_sems.at[idx],
                recv_sem=recv_sems.at[idx],
                device_id=(partner,),
                device_id_type=pl.DeviceIdType.MESH,
            )

        dma_b_low.wait()
        dma_a_1.wait()
        compute_low = make_compute(0)
        compute_low(ha_send + oa_send, hq)
        a1p1 = ex(ha_send + oa_send, hq, 0, xp,
                  recv_h.at[0, pl.ds(oa_send, hq), :], 0)
        pl.semaphore_wait(barrier_sem, 2)
        a1p1.start()
        compute_low(ha_send + oa_send + hq, hq)
        a1p2 = ex(ha_send + oa_send + hq, hq, 0, xp,
                  recv_h.at[0, pl.ds(oa_send + hq, hq), :], 9)
        a1p2.start()
        dma_b_high.wait()
        dma_a_2.wait()
        compute_high = make_compute(hw)
        compute_high(hb_send + ob_send, hq)
        b1p1 = ex(hb_send + ob_send, hq, hw, yp,
                  recv_h.at[1, pl.ds(ob_send, hq), :], 10)
        b1p1.start()
        compute_high(hb_send + ob_send + hq, hq)
        b1p2 = ex(hb_send + ob_send + hq, hq, hw, yp,
                  recv_h.at[1, pl.ds(ob_send + hq, hq), :], 19)
        b1p2.start()
        compute_low(ha_send + oa_keep, qm)
        a2 = ex(ha_send + oa_keep, qm, 0, xp,
                recv_h.at[0, pl.ds(oa_keep, qm), :], 1)
        a2.start()
        compute_high(hb_send + ob_keep, qm)
        b2 = ex(hb_send + ob_keep, qm, hw, yp,
                recv_h.at[1, pl.ds(ob_keep, qm), :], 11)
        b2.start()

        compute_low(qa_send, qm)
        compute_high(qb_send, qm)
        compute_low(qa_keep, qm)
        compute_high(qb_keep, qm)

        a1p1.wait()
        a1p2.wait()
        out_ref[pl.ds(qa_send, qm), pl.ds(0, hw)] += \
            recv_h[0, pl.ds(oa_send, qm), :]
        st2a = []
        for r in range(2):
            e = ex(qa_send + r * hq, hq, 0, yp,
                   recv_q.at[0, pl.ds(r * hq, hq), :], 2 + r)
            e.start()
            st2a.append(e)
        b1p1.wait()
        b1p2.wait()
        out_ref[pl.ds(qb_send, qm), pl.ds(hw, hw)] += \
            recv_h[1, pl.ds(ob_send, qm), :]
        st2b = []
        for r in range(2):
            e = ex(qb_send + r * hq, hq, hw, xp,
                   recv_q.at[1, pl.ds(r * hq, hq), :], 12 + r)
            e.start()
            st2b.append(e)
        a2.wait()
        out_ref[pl.ds(qa_keep, qm), pl.ds(0, hw)] += \
            recv_h[0, pl.ds(oa_keep, qm), :]
        b2.wait()
        out_ref[pl.ds(qb_keep, qm), pl.ds(hw, hw)] += \
            recv_h[1, pl.ds(ob_keep, qm), :]

        st3a = []
        for r in range(2):
            st2a[r].wait()
            out_ref[pl.ds(qa_keep + r * hq, hq), pl.ds(0, hw)] += \
                recv_q[0, pl.ds(r * hq, hq), :]
            e = ex(qa_keep + r * hq, hq, 0, yp,
                   out_ref.at[pl.ds(qa_keep + r * hq, hq), pl.ds(0, hw)],
                   4 + r)
            e.start()
            st3a.append(e)
        st4a1 = ex(qa_keep, qm, 0, xp,
                   out_hbm.at[pl.ds(qa_keep, qm), pl.ds(0, hw)], 6)
        st4a1.start()
        st3b = []
        for r in range(2):
            st2b[r].wait()
            out_ref[pl.ds(qb_keep + r * hq, hq), pl.ds(hw, hw)] += \
                recv_q[1, pl.ds(r * hq, hq), :]
            e = ex(qb_keep + r * hq, hq, hw, xp,
                   out_ref.at[pl.ds(qb_keep + r * hq, hq), pl.ds(hw, hw)],
                   14 + r)
            e.start()
            st3b.append(e)
        st4b1 = ex(qb_keep, qm, hw, yp,
                   out_hbm.at[pl.ds(qb_keep, qm), pl.ds(hw, hw)], 16)
        st4b1.start()

        st4a2 = []
        for r in range(2):
            st3a[r].wait()
            e = ex(qa_send + r * hq, hq, 0, xp,
                   out_hbm.at[pl.ds(qa_send + r * hq, hq), pl.ds(0, hw)],
                   7 + r)
            e.start()
            st4a2.append(e)
        dma_out_a = pltpu.make_async_copy(
            out_ref.at[pl.ds(ha_keep, hm), pl.ds(0, hw)],
            out_hbm.at[pl.ds(ha_keep, hm), pl.ds(0, hw)], out_sems.at[0])
        dma_out_a.start()
        st4b2 = []
        for r in range(2):
            st3b[r].wait()
            e = ex(qb_send + r * hq, hq, hw, yp,
                   out_hbm.at[pl.ds(qb_send + r * hq, hq), pl.ds(hw, hw)],
                   17 + r)
            e.start()
            st4b2.append(e)

        dma_out_b = pltpu.make_async_copy(
            out_ref.at[pl.ds(hb_keep, hm), pl.ds(hw, hw)],
            out_hbm.at[pl.ds(hb_keep, hm), pl.ds(hw, hw)], out_sems.at[1])
        dma_out_b.start()

        dma_out_a.wait()
        dma_out_b.wait()
        st4a1.wait()
        st4b1.wait()
        for e in st4a2 + st4b2:
            e.wait()

    return pl.pallas_call(
        body,
        out_shape=jax.ShapeDtypeStruct((m, n), jnp.bfloat16),
        in_specs=[
            pl.BlockSpec(memory_space=pltpu.MemorySpace.HBM),
            pl.BlockSpec(memory_space=pltpu.MemorySpace.HBM),
        ],
        out_specs=pl.BlockSpec(memory_space=pltpu.MemorySpace.HBM),
        scratch_shapes=[
            pltpu.VMEM((m, k), jnp.float32),
            pltpu.VMEM((k, n), jnp.float32),
            pltpu.SemaphoreType.DMA((4,)),
            pltpu.VMEM((m, n), jnp.bfloat16),
            pltpu.SemaphoreType.DMA((2,)),
            pltpu.VMEM((2, hm, hw), jnp.bfloat16),
            pltpu.VMEM((2, qm, hw), jnp.bfloat16),
            pltpu.SemaphoreType.DMA((20,)),
            pltpu.SemaphoreType.DMA((20,)),
        ],
        compiler_params=pltpu.CompilerParams(
            collective_id=0,
            vmem_limit_bytes=100 * 1024 * 1024,
        ),
    )(A, B)
